# Initial kernel scaffold; baseline (speedup 1.0000x reference)
import jax
import jax.numpy as jnp
from jax.experimental import pallas as pl


def kernel(tokens, token_table, pos_table):
    raise NotImplementedError("write your pallas kernel here")


# speedup vs baseline: 1.0228x; 1.0228x over previous
import functools

import jax
import jax.numpy as jnp
from jax import lax
from jax.experimental import pallas as pl
from jax.experimental.pallas import tpu as pltpu
from jax.experimental.pallas import tpu_sc as plsc


_LANES = 16


@functools.lru_cache(maxsize=None)
def _build_embed_kernel(T, V, D, L):
    info = plsc.get_sparse_core_info()
    NC, NS = info.num_cores, info.num_subcores
    NW = NC * NS
    assert T % NW == 0
    TPW = T // NW
    assert L % TPW == 0
    CH = 64
    assert TPW % CH == 0
    NCHUNK = TPW // CH
    assert D % _LANES == 0

    mesh = plsc.VectorSubcoreMesh(core_axis_name="c", subcore_axis_name="s")

    @functools.partial(
        pl.kernel,
        out_type=jax.ShapeDtypeStruct((T, D), jnp.float32),
        mesh=mesh,
        scratch_types=[
            pltpu.VMEM((CH,), jnp.int32),
            pltpu.VMEM((CH, D), jnp.float32),
            pltpu.VMEM((CH, D), jnp.float32),
            pltpu.SemaphoreType.DMA,
        ],
    )
    def embed(tok_hbm, tab_hbm, pos_hbm, out_hbm, idx_v, pos_v, rows_v, sem):
        wid = lax.axis_index("s") * NC + lax.axis_index("c")
        base = wid * TPW
        l0 = base % L

        @pl.loop(0, NCHUNK)
        def _chunk(c):
            off = c * CH
            pltpu.sync_copy(tok_hbm.at[pl.ds(base + off, CH)], idx_v)
            pltpu.sync_copy(pos_hbm.at[pl.ds(l0 + off, CH)], pos_v)
            pltpu.async_copy(tab_hbm.at[idx_v], rows_v, sem).wait()

            @pl.loop(0, CH)
            def _row(r):
                for j in range(D // _LANES):
                    sl = pl.ds(j * _LANES, _LANES)
                    rows_v[r, sl] = rows_v[r, sl] + pos_v[r, sl]

            pltpu.sync_copy(rows_v, out_hbm.at[pl.ds(base + off, CH)])

    return embed


def kernel(tokens, token_table, pos_table):
    B, L = tokens.shape
    V, D = token_table.shape
    embed = _build_embed_kernel(B * L, V, D, L)
    out = embed(tokens.reshape(-1), token_table, pos_table)
    return out.reshape(B, L, D)


# speedup vs baseline: 1.2262x; 1.1989x over previous
import functools

import jax
import jax.numpy as jnp
from jax import lax
from jax.experimental import pallas as pl
from jax.experimental.pallas import tpu as pltpu
from jax.experimental.pallas import tpu_sc as plsc


_LANES = 16


@functools.lru_cache(maxsize=None)
def _build_embed_kernel(T, V, D, L):
    info = plsc.get_sparse_core_info()
    NC, NS = info.num_cores, info.num_subcores
    NW = NC * NS
    assert T % NW == 0
    TPW = T // NW
    assert L % TPW == 0
    CH = 32
    assert TPW % CH == 0
    NCHUNK = TPW // CH
    assert D % _LANES == 0

    mesh = plsc.VectorSubcoreMesh(core_axis_name="c", subcore_axis_name="s")

    @functools.partial(
        pl.kernel,
        out_type=jax.ShapeDtypeStruct((T, D), jnp.float32),
        mesh=mesh,
        scratch_types=[
            pltpu.VMEM((TPW,), jnp.int32),
            [pltpu.VMEM((CH, D), jnp.float32) for _ in range(2)],
            [pltpu.VMEM((CH, D), jnp.float32) for _ in range(2)],
            [pltpu.SemaphoreType.DMA for _ in range(2)],
            [pltpu.SemaphoreType.DMA for _ in range(2)],
            [pltpu.SemaphoreType.DMA for _ in range(2)],
        ],
    )
    def embed(tok_hbm, tab_hbm, pos_hbm, out_hbm,
              idx_v, pos_v, rows_v, psem, gsem, osem):
        wid = lax.axis_index("s") * NC + lax.axis_index("c")
        base = wid * TPW
        l0 = base % L

        pltpu.sync_copy(tok_hbm.at[pl.ds(base, TPW)], idx_v)

        def start(c, s):
            pd = pltpu.async_copy(
                pos_hbm.at[pl.ds(l0 + c * CH, CH)], pos_v[s], psem[s])
            gd = pltpu.async_copy(
                tab_hbm.at[idx_v.at[pl.ds(c * CH, CH)]], rows_v[s], gsem[s])
            return pd, gd

        in_d = [None, None]
        out_d = [None, None]
        in_d[0] = start(0, 0)
        for c in range(NCHUNK):
            s = c & 1
            n = s ^ 1
            if c + 1 < NCHUNK:
                if out_d[n] is not None:
                    out_d[n].wait()
                in_d[n] = start(c + 1, n)
            in_d[s][0].wait()
            in_d[s][1].wait()

            @pl.loop(0, CH)
            def _row(r):
                for j in range(D // _LANES):
                    sl = pl.ds(j * _LANES, _LANES)
                    rows_v[s][r, sl] = rows_v[s][r, sl] + pos_v[s][r, sl]

            out_d[s] = pltpu.async_copy(
                rows_v[s], out_hbm.at[pl.ds(base + c * CH, CH)], osem[s])
        out_d[0].wait()
        out_d[1].wait()

    return embed


def kernel(tokens, token_table, pos_table):
    B, L = tokens.shape
    V, D = token_table.shape
    embed = _build_embed_kernel(B * L, V, D, L)
    out = embed(tokens.reshape(-1), token_table, pos_table)
    return out.reshape(B, L, D)
